# Initial kernel scaffold; baseline (speedup 1.0000x reference)
import jax
import jax.numpy as jnp
from jax.experimental import pallas as pl


def kernel(x, ls_indices, W1, b1, W1f, b1f, W2, b2, W3, b3):
    raise NotImplementedError("write your pallas kernel here")


# speedup vs baseline: 9.1035x; 9.1035x over previous
import jax
import jax.numpy as jnp
from jax.experimental import pallas as pl
from jax.experimental.pallas import tpu as pltpu

COUNT = 8
L1 = 1024
L2 = 15
L3 = 32
SQR_C = 0.9921875

TB = 512


def _body(idx_ref, x_ref, w1_ref, w1f_ref, b1_ref, b1f_ref,
          w2t_ref, b2_ref, w3t_ref, b3_ref, out_ref):
    G = L2 + 1
    xb = x_ref[...]
    w1f_big = jnp.concatenate([w1f_ref[...]] * COUNT, axis=0)
    weff = w1_ref[...] + w1f_big
    b1f_big = jnp.concatenate([b1f_ref[...]] * COUNT, axis=1)
    beff = b1_ref[...] + b1f_big
    y = jax.lax.dot_general(xb, weff, (((1,), (1,)), ((), ())),
                            preferred_element_type=jnp.float32) + beff
    idx = idx_ref[...]
    sel = jnp.where(idx == 0, y[:, 0:G], 0.0)
    for k in range(1, COUNT):
        sel = sel + jnp.where(idx == k, y[:, G * k:G * (k + 1)], 0.0)
    first = sel[:, :L2]
    out1 = sel[:, L2:G]
    sqr = first * first * SQR_C
    l1x = jnp.clip(jnp.concatenate([sqr, first], axis=1), 0.0, 1.0)
    l2all = jax.lax.dot_general(l1x, w2t_ref[...], (((1,), (0,)), ((), ())),
                                preferred_element_type=jnp.float32) + b2_ref[...]
    sel2 = jnp.where(idx == 0, l2all[:, 0:L3], 0.0)
    for k in range(1, COUNT):
        sel2 = sel2 + jnp.where(idx == k, l2all[:, L3 * k:L3 * (k + 1)], 0.0)
    l2x = jnp.clip(sel2, 0.0, 1.0)
    l3all = jax.lax.dot_general(l2x, w3t_ref[...], (((1,), (0,)), ((), ())),
                                preferred_element_type=jnp.float32) + b3_ref[...]
    sel3 = jnp.where(idx == 0, l3all[:, 0:1], 0.0)
    for k in range(1, COUNT):
        sel3 = sel3 + jnp.where(idx == k, l3all[:, k:k + 1], 0.0)
    out_ref[...] = sel3 + out1


def kernel(x, ls_indices, W1, b1, W1f, b1f, W2, b2, W3, b3):
    B = x.shape[0]
    G = L2 + 1
    idx = ls_indices.astype(jnp.int32).reshape(B, 1)
    W1r = W1.reshape(COUNT * G, L1)
    b1r = b1.reshape(1, COUNT * G)
    b1fr = b1f.reshape(1, G)
    W2T = jnp.transpose(W2, (2, 0, 1)).reshape(2 * L2, COUNT * L3)
    b2r = b2.reshape(1, COUNT * L3)
    W3T = W3.reshape(COUNT, L3).T
    b3r = b3.reshape(1, COUNT)

    grid = (B // TB,)
    full = lambda i: (0, 0)
    out = pl.pallas_call(
        _body,
        grid=grid,
        in_specs=[
            pl.BlockSpec((TB, 1), lambda i: (i, 0)),
            pl.BlockSpec((TB, L1), lambda i: (i, 0)),
            pl.BlockSpec((COUNT * G, L1), full),
            pl.BlockSpec((G, L1), full),
            pl.BlockSpec((1, COUNT * G), full),
            pl.BlockSpec((1, G), full),
            pl.BlockSpec((2 * L2, COUNT * L3), full),
            pl.BlockSpec((1, COUNT * L3), full),
            pl.BlockSpec((L3, COUNT), full),
            pl.BlockSpec((1, COUNT), full),
        ],
        out_specs=pl.BlockSpec((TB, 1), lambda i: (i, 0)),
        out_shape=jax.ShapeDtypeStruct((B, 1), jnp.float32),
        compiler_params=pltpu.CompilerParams(
            dimension_semantics=("arbitrary",),
        ),
    )(idx, x, W1r, W1f.reshape(G, L1), b1r, b1fr, W2T, b2r, W3T, b3r)
    return out


# speedup vs baseline: 11.9645x; 1.3143x over previous
import functools

import jax
import jax.numpy as jnp
from jax import lax
from jax.experimental import pallas as pl
from jax.experimental.pallas import tpu as pltpu
from jax.experimental.pallas import tpu_sc as plsc

COUNT = 8
L1 = 1024
L2 = 15
L3 = 32
G = L2 + 1
SQR_C = 0.9921875

TB = 512

def _tc_body(x_ref, w1_ref, w1f_ref, b1_ref, b1f_ref,
             w2big_ref, b2_ref, w3big_ref, b3_ref,
             out_ref, weff_ref, beff_ref):
    @pl.when(pl.program_id(0) == 0)
    def _():
        w1f_big = jnp.concatenate([w1f_ref[...]] * COUNT, axis=0)
        weff_ref[...] = w1_ref[...] + w1f_big
        b1f_big = jnp.concatenate([b1f_ref[...]] * COUNT, axis=1)
        beff_ref[...] = b1_ref[...] + b1f_big

    xb = x_ref[...]
    y = jax.lax.dot_general(xb, weff_ref[...], (((1,), (1,)), ((), ())),
                            preferred_element_type=jnp.float32) + beff_ref[...]
    act = jnp.concatenate(
        [jnp.clip(y * y * SQR_C, 0.0, 1.0), jnp.clip(y, 0.0, 1.0)], axis=1)
    l2 = jax.lax.dot_general(act, w2big_ref[...], (((1,), (0,)), ((), ())),
                             preferred_element_type=jnp.float32) + b2_ref[...]
    l2x = jnp.clip(l2, 0.0, 1.0)
    l3 = jax.lax.dot_general(l2x, w3big_ref[...], (((1,), (0,)), ((), ())),
                             preferred_element_type=jnp.float32)
    skip = jnp.concatenate(
        [y[:, G * k + L2:G * k + G] for k in range(COUNT)], axis=1)
    out_ref[...] = l3 + skip + b3_ref[...]


def _tc_scores(x, W1r, W1f, b1r, b1fr, W2big, b2r, W3big, b3r):
    B = x.shape[0]
    full = lambda i: (0, 0)
    return pl.pallas_call(
        _tc_body,
        grid=(B // TB,),
        in_specs=[
            pl.BlockSpec((TB, L1), lambda i: (i, 0)),
            pl.BlockSpec((COUNT * G, L1), full),
            pl.BlockSpec((G, L1), full),
            pl.BlockSpec((1, COUNT * G), full),
            pl.BlockSpec((1, G), full),
            pl.BlockSpec((2 * COUNT * G, COUNT * L3), full),
            pl.BlockSpec((1, COUNT * L3), full),
            pl.BlockSpec((COUNT * L3, COUNT), full),
            pl.BlockSpec((1, COUNT), full),
        ],
        out_specs=pl.BlockSpec((TB, COUNT), lambda i: (i, 0)),
        out_shape=jax.ShapeDtypeStruct((B, COUNT), jnp.float32),
        scratch_shapes=[
            pltpu.VMEM((COUNT * G, L1), jnp.float32),
            pltpu.VMEM((1, COUNT * G), jnp.float32),
        ],
        compiler_params=pltpu.CompilerParams(
            dimension_semantics=("arbitrary",),
        ),
    )(x, W1r, W1f, b1r, b1fr, W2big, b2r, W3big, b3r)


def _make_sc_gather(B):
    NC, NS, L = 2, 16, 16
    NW = NC * NS
    bpw = B // NW
    mesh = plsc.VectorSubcoreMesh(core_axis_name="c", subcore_axis_name="s")

    CH = 128
    NCH = bpw // CH

    @functools.partial(
        pl.kernel, mesh=mesh,
        out_type=jax.ShapeDtypeStruct((B,), jnp.float32),
        scratch_types=[
            pltpu.VMEM((bpw,), jnp.int32),
            pltpu.VMEM((NCH, CH), jnp.int32),
            pltpu.VMEM((bpw,), jnp.float32),
            pltpu.SemaphoreType.DMA,
        ],
    )
    def sc_gather(scores_hbm, idx_hbm, out_hbm, idx_v, fidx_v, out_v, sem):
        wid = lax.axis_index("s") * NC + lax.axis_index("c")
        base = wid * bpw
        pltpu.sync_copy(idx_hbm.at[pl.ds(base, bpw)], idx_v)
        per_row = CH // L
        for i in range(bpw // L):
            lane = lax.iota(jnp.int32, L)
            flat = (lane + (base + i * L)) * COUNT + idx_v[pl.ds(i * L, L)]
            fidx_v[i // per_row, pl.ds((i % per_row) * L, L)] = flat
        for j in range(NCH):
            pltpu.async_copy(scores_hbm.at[fidx_v.at[j]],
                             out_v.at[pl.ds(j * CH, CH)], sem).wait()
        pltpu.sync_copy(out_v, out_hbm.at[pl.ds(base, bpw)])

    return sc_gather


def kernel(x, ls_indices, W1, b1, W1f, b1f, W2, b2, W3, b3):
    B = x.shape[0]
    idx = ls_indices.astype(jnp.int32)
    W1r = W1.reshape(COUNT * G, L1)
    b1r = b1.reshape(1, COUNT * G)
    b1fr = b1f.reshape(1, G)
    eye = jnp.eye(COUNT, dtype=W2.dtype)
    W2sqr = jnp.pad(jnp.transpose(W2[:, :, :L2], (0, 2, 1)),
                    ((0, 0), (0, 1), (0, 0)))
    W2raw = jnp.pad(jnp.transpose(W2[:, :, L2:], (0, 2, 1)),
                    ((0, 0), (0, 1), (0, 0)))
    top = jnp.einsum('kjo,kK->kjKo', W2sqr, eye).reshape(COUNT * G, COUNT * L3)
    bot = jnp.einsum('kjo,kK->kjKo', W2raw, eye).reshape(COUNT * G, COUNT * L3)
    W2big = jnp.concatenate([top, bot], axis=0)
    b2r = b2.reshape(1, COUNT * L3)
    W3big = jnp.einsum('ko,kK->koK', W3.reshape(COUNT, L3),
                       eye).reshape(COUNT * L3, COUNT)
    b3r = b3.reshape(1, COUNT)

    scores = _tc_scores(x, W1r, W1f, b1r, b1fr, W2big, b2r, W3big, b3r)
    out = _make_sc_gather(B)(scores.reshape(B * COUNT), idx)
    return out.reshape(B, 1)
